# baseline (speedup 1.0000x reference)
import functools

import jax
import jax.numpy as jnp
from jax import lax
from jax.experimental import pallas as pl
from jax.experimental.pallas import tpu as pltpu
from jax.experimental.pallas import tpu_sc as plsc

N = 10000
F = 128
H = 64
C = 40
K = 20
NB = 8

NPAD = 10240
E_PAD = 163840
NW = 32

def _sc_mesh():
    return plsc.VectorSubcoreMesh(core_axis_name="c", subcore_axis_name="s")


def _elu(h_p):
    def body(h_ref, o_ref):
        he = h_ref[...]
        o_ref[...] = jnp.where(he > 0, he, jnp.exp(he) - 1.0)

    return pl.pallas_call(
        body,
        out_shape=jax.ShapeDtypeStruct((NPAD, H), jnp.float32),
    )(h_p)


def _sc_gather(table, ids, gb):
    m = ids.shape[0]
    d = table.shape[1]
    per_w = m // NW
    nb = per_w // gb

    @functools.partial(
        pl.kernel,
        out_type=jax.ShapeDtypeStruct((m, d), jnp.float32),
        mesh=_sc_mesh(),
        compiler_params=pltpu.CompilerParams(use_tc_tiling_on_sc=False),
        scratch_types=[
            pltpu.VMEM((2, gb), jnp.int32),
            pltpu.VMEM((2, gb, d), jnp.float32),
            pltpu.SemaphoreType.DMA,
            pltpu.SemaphoreType.DMA,
            pltpu.SemaphoreType.DMA,
            pltpu.SemaphoreType.DMA,
        ],
    )
    def k(table_hbm, ids_hbm, out_hbm, idx_v, rows_v, gsa, gsb, wsa, wsb):
        wid = lax.axis_index("s") * 2 + lax.axis_index("c")
        base = wid * per_w

        def load_idx(i, buf):
            pltpu.sync_copy(ids_hbm.at[pl.ds(base + i * gb, gb)], idx_v.at[buf])

        load_idx(0, 0)
        pltpu.async_copy(table_hbm.at[idx_v.at[0]], rows_v.at[0], gsa)
        load_idx(1, 1)
        pltpu.async_copy(table_hbm.at[idx_v.at[1]], rows_v.at[1], gsb)

        @pl.loop(0, nb, step=2)
        def _(i):
            pltpu.make_async_copy(table_hbm.at[idx_v.at[0]], rows_v.at[0], gsa).wait()
            pltpu.async_copy(rows_v.at[0], out_hbm.at[pl.ds(base + i * gb, gb)], wsa)
            pltpu.make_async_copy(table_hbm.at[idx_v.at[1]], rows_v.at[1], gsb).wait()
            pltpu.async_copy(rows_v.at[1], out_hbm.at[pl.ds(base + (i + 1) * gb, gb)], wsb)

            @pl.when(i + 2 < nb)
            def _():
                load_idx(i + 2, 0)
                pltpu.make_async_copy(rows_v.at[0], out_hbm.at[pl.ds(0, gb)], wsa).wait()
                pltpu.async_copy(table_hbm.at[idx_v.at[0]], rows_v.at[0], gsa)

            @pl.when(i + 3 < nb)
            def _():
                load_idx(i + 3, 1)
                pltpu.make_async_copy(rows_v.at[1], out_hbm.at[pl.ds(0, gb)], wsb).wait()
                pltpu.async_copy(table_hbm.at[idx_v.at[1]], rows_v.at[1], gsb)

        pltpu.make_async_copy(rows_v.at[0], out_hbm.at[pl.ds(0, gb)], wsa).wait()
        pltpu.make_async_copy(rows_v.at[1], out_hbm.at[pl.ds(0, gb)], wsb).wait()

    return k(table, ids)


def _mlp_edge(gathered, w1, w2, b1r, b2r):
    TB = 2048
    nt = E_PAD // TB

    def body(xi_ref, xj_ref, w1_ref, w2_ref, b1_ref, b2_ref, o_ref):
        xi = xi_ref[...]
        cat = jnp.concatenate([xi, xj_ref[...] - xi], axis=1)
        h1 = jnp.dot(cat, w1_ref[...], preferred_element_type=jnp.float32) + b1_ref[...]
        h1 = jnp.maximum(h1, 0.0)
        h2 = jnp.dot(h1, w2_ref[...], preferred_element_type=jnp.float32) + b2_ref[...]
        o_ref[...] = jnp.maximum(h2, 0.0)

    return pl.pallas_call(
        body,
        grid=(nt,),
        in_specs=[
            pl.BlockSpec((TB, F), lambda i: (i, 0)),
            pl.BlockSpec((TB, F), lambda i: (i + nt, 0)),
            pl.BlockSpec((2 * F, H), lambda i: (0, 0)),
            pl.BlockSpec((H, H), lambda i: (0, 0)),
            pl.BlockSpec((1, H), lambda i: (0, 0)),
            pl.BlockSpec((1, H), lambda i: (0, 0)),
        ],
        out_specs=pl.BlockSpec((TB, H), lambda i: (i, 0)),
        out_shape=jax.ShapeDtypeStruct((E_PAD, H), jnp.float32),
    )(gathered, gathered, w1, w2, b1r, b2r)


_R = NPAD // NW
_SC_CH = 16384
_GB_S = 256
_NSUP = E_PAD // _SC_CH


def _sc_scatter_max(m, dst_s):
    @functools.partial(
        pl.kernel,
        out_type=jax.ShapeDtypeStruct((NPAD * H,), jnp.float32),
        mesh=_sc_mesh(),
        compiler_params=pltpu.CompilerParams(
            use_tc_tiling_on_sc=False, needs_layout_passes=False),
        scratch_types=[
            pltpu.VMEM((_SC_CH,), jnp.int32),
            pltpu.VMEM((_SC_CH + 16,), jnp.int32),
            pltpu.VMEM((_SC_CH + 16,), jnp.int32),
            pltpu.VMEM((2, _GB_S, H), jnp.float32),
            pltpu.VMEM((_R * H,), jnp.float32),
            pltpu.SemaphoreType.DMA,
        ],
    )
    def k(m_hbm, dst_hbm, out_hbm, dstbuf, idbuf, dstcbuf, rows_v, acc, sem):
        wid = lax.axis_index("s") * 2 + lax.axis_index("c")
        lo = wid * _R
        zf16 = jnp.zeros((16,), jnp.float32)
        zi16 = jnp.zeros((16,), jnp.int32)

        @pl.loop(0, _R * H // 16)
        def _(i):
            acc[pl.ds(i * 16, 16)] = zf16

        @pl.loop(0, (_SC_CH + 16) // 16)
        def _(i):
            idbuf[pl.ds(i * 16, 16)] = zi16

        iota16 = lax.broadcasted_iota(jnp.int32, (16,), 0)

        @pl.loop(0, _NSUP)
        def _(s):
            ebase = s * _SC_CH
            pltpu.sync_copy(dst_hbm.at[pl.ds(ebase, _SC_CH)], dstbuf)

            def scan_body(g, cnt):
                v = dstbuf[pl.ds(g * 16, 16)]
                mask = (v >= lo) & (v < lo + _R)
                ids = ebase + g * 16 + iota16
                plsc.store_compressed(idbuf.at[pl.ds(cnt, 16)], ids, mask=mask)
                plsc.store_compressed(dstcbuf.at[pl.ds(cnt, 16)], v, mask=mask)
                c = plsc.all_reduce_population_count(mask)
                return cnt + jnp.max(c)

            cnt = lax.fori_loop(0, _SC_CH // 16, scan_body, jnp.int32(0), unroll=4)

            nbat = (cnt + (_GB_S - 1)) // _GB_S

            @pl.when(nbat > 0)
            def _():
                pltpu.async_copy(
                    m_hbm.at[idbuf.at[pl.ds(0, _GB_S)]], rows_v.at[0], sem)

            def batch_body(gi, _):
                b = gi % 2
                goff = gi * _GB_S
                pltpu.make_async_copy(
                    m_hbm.at[idbuf.at[pl.ds(goff, _GB_S)]], rows_v.at[b], sem
                ).wait()

                @pl.when(gi + 1 < nbat)
                def _():
                    pltpu.async_copy(
                        m_hbm.at[idbuf.at[pl.ds(goff + _GB_S, _GB_S)]],
                        rows_v.at[1 - b], sem)

                nloc = jnp.minimum(_GB_S, cnt - goff)

                def acc_body(e, __):
                    d = dstcbuf[pl.ds(goff + e, 16)][0]
                    rel = (d - lo) * H
                    for c2 in range(H // 16):
                        off = rel + c2 * 16
                        a = acc[pl.ds(off, 16)]
                        r = rows_v[b, e, pl.ds(c2 * 16, 16)]
                        acc[pl.ds(off, 16)] = jnp.maximum(a, r)
                    return 0

                lax.fori_loop(0, nloc, acc_body, 0)
                return 0

            lax.fori_loop(0, nbat, batch_body, 0)

        pltpu.sync_copy(acc, out_hbm.at[pl.ds(wid * _R * H, _R * H)])

    return k(m, dst_s).reshape(NPAD, H)


_RT = 1024
_CT = 1024
_KPAD = 32


def _knn(h_p, bc, br):
    nrt = NPAD // _RT

    def body(h_ref, bc_ref, br_ref, o_ref, buf_ref):
        INF = jnp.float32(jnp.inf)
        i = pl.program_id(0)
        hr = h_ref[pl.ds(i * _RT, _RT), :]
        sqr = jnp.sum(hr * hr, axis=1, keepdims=True)
        bcr = bc_ref[pl.ds(i * _RT, _RT), :]

        start_r = jnp.zeros((_RT, 1), jnp.int32)
        end_r = jnp.zeros((_RT, 1), jnp.int32)
        brow = br_ref[...]
        for b in range(NB):
            cb = jnp.sum(jnp.where(brow == b, 1, 0)).astype(jnp.int32)
            start_r = start_r + jnp.where(bcr > b, cb, 0)
            end_r = end_r + jnp.where(bcr >= b, cb, 0)
        tile_lo = jnp.min(start_r) // _CT
        tile_hi = (jnp.max(end_r) + _CT - 1) // _CT

        def dloop(t, _):
            hc = h_ref[pl.ds(t * _CT, _CT), :]
            sqc = jnp.sum(hc * hc, axis=1)[None, :]
            dot = lax.dot_general(
                hr, hc, (((1,), (1,)), ((), ())),
                preferred_element_type=jnp.float32,
            )
            d = (sqr + sqc) - 2.0 * dot
            mask = bcr == br_ref[pl.ds(0, 1), pl.ds(t * _CT, _CT)]
            buf_ref[:, pl.ds(t * _CT, _CT)] = jnp.where(mask, d, INF)
            return 0

        lax.fori_loop(tile_lo, tile_hi, dloop, 0)

        iota_c = lax.convert_element_type(
            lax.broadcasted_iota(jnp.int32, (_RT, _CT), 1), jnp.float32)
        BIGF = jnp.float32(2 * NPAD)
        prev = jnp.full((_RT, 1), BIGF)
        for kk in range(K):
            def eloop(t, carry):
                cm, ca = carry
                gidx = iota_c + lax.convert_element_type(t * _CT, jnp.float32)
                vals = buf_ref[:, pl.ds(t * _CT, _CT)]
                vals = jnp.where(gidx == prev, INF, vals)
                buf_ref[:, pl.ds(t * _CT, _CT)] = vals
                tmin = jnp.min(vals, axis=1, keepdims=True)
                targ = jnp.min(jnp.where(vals == tmin, gidx, BIGF),
                               axis=1, keepdims=True)
                upd = tmin < cm
                return (jnp.where(upd, tmin, cm), jnp.where(upd, targ, ca))

            cm, ca = lax.fori_loop(
                tile_lo, tile_hi, eloop,
                (jnp.full((_RT, 1), INF), jnp.full((_RT, 1), BIGF)),
            )
            o_ref[:, pl.ds(kk, 1)] = jnp.clip(
                lax.convert_element_type(ca, jnp.int32), 0, N - 1)
            prev = ca

    return pl.pallas_call(
        body,
        grid=(nrt,),
        in_specs=[
            pl.BlockSpec((NPAD, H), lambda i: (0, 0)),
            pl.BlockSpec((NPAD, 1), lambda i: (0, 0)),
            pl.BlockSpec((1, NPAD), lambda i: (0, 0)),
        ],
        out_specs=pl.BlockSpec((_RT, _KPAD), lambda i: (i, 0)),
        out_shape=jax.ShapeDtypeStruct((NPAD, _KPAD), jnp.int32),
        scratch_shapes=[pltpu.VMEM((_RT, NPAD), jnp.float32)],
    )(h_p, bc, br)


def _dyn_conv(g3, he, w1, w2, b1r, b2r, lwp, lbp, final):
    TB = 1024
    nt = NPAD // TB
    outw = 128 if final else H

    def body(g_ref, he_ref, w1_ref, w2_ref, b1_ref, b2_ref, lw_ref, lb_ref,
             o_ref, acc_ref):
        kk = pl.program_id(1)
        hi = he_ref[...]
        cat = jnp.concatenate([hi, g_ref[0] - hi], axis=1)
        h1 = jnp.dot(cat, w1_ref[...], preferred_element_type=jnp.float32) + b1_ref[...]
        h1 = jnp.maximum(h1, 0.0)
        h2 = jnp.dot(h1, w2_ref[...], preferred_element_type=jnp.float32) + b2_ref[...]
        h2 = jnp.maximum(h2, 0.0)

        @pl.when(kk == 0)
        def _():
            acc_ref[...] = h2

        @pl.when(kk > 0)
        def _():
            acc_ref[...] = jnp.maximum(acc_ref[...], h2)

        @pl.when(kk == K - 1)
        def _():
            if final:
                o_ref[...] = (
                    jnp.dot(acc_ref[...], lw_ref[...], preferred_element_type=jnp.float32)
                    + lb_ref[...]
                )
            else:
                o_ref[...] = acc_ref[...]

    return pl.pallas_call(
        body,
        grid=(nt, K),
        in_specs=[
            pl.BlockSpec((1, TB, H), lambda i, k: (k, i, 0)),
            pl.BlockSpec((TB, H), lambda i, k: (i, 0)),
            pl.BlockSpec((2 * H, H), lambda i, k: (0, 0)),
            pl.BlockSpec((H, H), lambda i, k: (0, 0)),
            pl.BlockSpec((1, H), lambda i, k: (0, 0)),
            pl.BlockSpec((1, H), lambda i, k: (0, 0)),
            pl.BlockSpec((H, 128), lambda i, k: (0, 0)),
            pl.BlockSpec((1, 128), lambda i, k: (0, 0)),
        ],
        out_specs=pl.BlockSpec((TB, outw), lambda i, k: (i, 0)),
        out_shape=jax.ShapeDtypeStruct((NPAD, outw), jnp.float32),
        scratch_shapes=[pltpu.VMEM((TB, H), jnp.float32)],
    )(g3, he, w1, w2, b1r, b2r, lwp, lbp)


def kernel(x, edge_index, batch, c1_W1, c1_b1, c1_W2, c1_b2,
           d1_W1, d1_b1, d1_W2, d1_b2, d2_W1, d2_b1, d2_W2, d2_b2,
           lin_W, lin_b):
    f32 = jnp.float32
    i32 = jnp.int32
    src = edge_index[0]
    dst = edge_index[1]
    e = src.shape[0]

    x_p = jnp.concatenate([x, jnp.zeros((NPAD - N, F), f32)], axis=0)
    batch_p = jnp.concatenate([batch.astype(i32), jnp.full((NPAD - N,), NB, i32)])
    bc = batch_p[:, None]
    br = batch_p[None, :]

    zpad = jnp.zeros((E_PAD - e,), i32)
    ids1 = jnp.concatenate([
        jnp.concatenate([dst, zpad]),
        jnp.concatenate([src, zpad]),
    ])
    gath1 = _sc_gather(x_p, ids1, 256)
    m1 = _mlp_edge(gath1, c1_W1, c1_W2, c1_b1.reshape(1, H), c1_b2.reshape(1, H))
    dst_s = jnp.concatenate([dst, jnp.full((E_PAD - e,), NPAD, i32)])
    h = _sc_scatter_max(m1, dst_s)

    lwp = jnp.concatenate([lin_W, jnp.zeros((H, 128 - C), f32)], axis=1)
    lbp = jnp.concatenate([lin_b, jnp.zeros((128 - C,), f32)]).reshape(1, 128)

    layers = ((d1_W1, d1_b1, d1_W2, d1_b2, False),
              (d2_W1, d2_b1, d2_W2, d2_b2, True))
    for (w1, b1, w2, b2, final) in layers:
        he = _elu(h)
        idxk = _knn(he, bc, br)
        ids2 = idxk[:, :K].T.reshape(K * NPAD)
        g = _sc_gather(he, ids2, 640)
        g3 = g.reshape(K, NPAD, H)
        h = _dyn_conv(g3, he, w1, w2, b1.reshape(1, H), b2.reshape(1, H),
                      lwp, lbp, final)

    return h[:N, :C]


# speedup vs baseline: 1.0408x; 1.0408x over previous
import functools

import jax
import jax.numpy as jnp
from jax import lax
from jax.experimental import pallas as pl
from jax.experimental.pallas import tpu as pltpu
from jax.experimental.pallas import tpu_sc as plsc

N = 10000
F = 128
H = 64
C = 40
K = 20
NB = 8

NPAD = 10240
E_PAD = 163840
NW = 32

def _sc_mesh():
    return plsc.VectorSubcoreMesh(core_axis_name="c", subcore_axis_name="s")


def _elu(h_p):
    def body(h_ref, o_ref):
        he = h_ref[...]
        o_ref[...] = jnp.where(he > 0, he, jnp.exp(he) - 1.0)

    return pl.pallas_call(
        body,
        out_shape=jax.ShapeDtypeStruct((NPAD, H), jnp.float32),
    )(h_p)


def _sc_gather(table, ids, gb):
    m = ids.shape[0]
    d = table.shape[1]
    per_w = m // NW
    nb = per_w // gb

    @functools.partial(
        pl.kernel,
        out_type=jax.ShapeDtypeStruct((m, d), jnp.float32),
        mesh=_sc_mesh(),
        compiler_params=pltpu.CompilerParams(use_tc_tiling_on_sc=False),
        scratch_types=[
            pltpu.VMEM((2, gb), jnp.int32),
            pltpu.VMEM((2, gb, d), jnp.float32),
            pltpu.SemaphoreType.DMA,
            pltpu.SemaphoreType.DMA,
            pltpu.SemaphoreType.DMA,
            pltpu.SemaphoreType.DMA,
        ],
    )
    def k(table_hbm, ids_hbm, out_hbm, idx_v, rows_v, gsa, gsb, wsa, wsb):
        wid = lax.axis_index("s") * 2 + lax.axis_index("c")
        base = wid * per_w

        def load_idx(i, buf):
            pltpu.sync_copy(ids_hbm.at[pl.ds(base + i * gb, gb)], idx_v.at[buf])

        load_idx(0, 0)
        pltpu.async_copy(table_hbm.at[idx_v.at[0]], rows_v.at[0], gsa)
        load_idx(1, 1)
        pltpu.async_copy(table_hbm.at[idx_v.at[1]], rows_v.at[1], gsb)

        @pl.loop(0, nb, step=2)
        def _(i):
            pltpu.make_async_copy(table_hbm.at[idx_v.at[0]], rows_v.at[0], gsa).wait()
            pltpu.async_copy(rows_v.at[0], out_hbm.at[pl.ds(base + i * gb, gb)], wsa)
            pltpu.make_async_copy(table_hbm.at[idx_v.at[1]], rows_v.at[1], gsb).wait()
            pltpu.async_copy(rows_v.at[1], out_hbm.at[pl.ds(base + (i + 1) * gb, gb)], wsb)

            @pl.when(i + 2 < nb)
            def _():
                load_idx(i + 2, 0)
                pltpu.make_async_copy(rows_v.at[0], out_hbm.at[pl.ds(0, gb)], wsa).wait()
                pltpu.async_copy(table_hbm.at[idx_v.at[0]], rows_v.at[0], gsa)

            @pl.when(i + 3 < nb)
            def _():
                load_idx(i + 3, 1)
                pltpu.make_async_copy(rows_v.at[1], out_hbm.at[pl.ds(0, gb)], wsb).wait()
                pltpu.async_copy(table_hbm.at[idx_v.at[1]], rows_v.at[1], gsb)

        pltpu.make_async_copy(rows_v.at[0], out_hbm.at[pl.ds(0, gb)], wsa).wait()
        pltpu.make_async_copy(rows_v.at[1], out_hbm.at[pl.ds(0, gb)], wsb).wait()

    return k(table, ids)


def _mlp_edge(gathered, w1, w2, b1r, b2r):
    TB = 2048
    nt = E_PAD // TB

    def body(xi_ref, xj_ref, w1_ref, w2_ref, b1_ref, b2_ref, o_ref):
        xi = xi_ref[...]
        cat = jnp.concatenate([xi, xj_ref[...] - xi], axis=1)
        h1 = jnp.dot(cat, w1_ref[...], preferred_element_type=jnp.float32) + b1_ref[...]
        h1 = jnp.maximum(h1, 0.0)
        h2 = jnp.dot(h1, w2_ref[...], preferred_element_type=jnp.float32) + b2_ref[...]
        o_ref[...] = jnp.maximum(h2, 0.0)

    return pl.pallas_call(
        body,
        grid=(nt,),
        in_specs=[
            pl.BlockSpec((TB, F), lambda i: (i, 0)),
            pl.BlockSpec((TB, F), lambda i: (i + nt, 0)),
            pl.BlockSpec((2 * F, H), lambda i: (0, 0)),
            pl.BlockSpec((H, H), lambda i: (0, 0)),
            pl.BlockSpec((1, H), lambda i: (0, 0)),
            pl.BlockSpec((1, H), lambda i: (0, 0)),
        ],
        out_specs=pl.BlockSpec((TB, H), lambda i: (i, 0)),
        out_shape=jax.ShapeDtypeStruct((E_PAD, H), jnp.float32),
    )(gathered, gathered, w1, w2, b1r, b2r)


_R = NPAD // NW
_SC_CH = 16384
_GB_S = 256
_NSUP = E_PAD // _SC_CH


def _sc_scatter_max(m, dst_s):
    @functools.partial(
        pl.kernel,
        out_type=jax.ShapeDtypeStruct((NPAD * H,), jnp.float32),
        mesh=_sc_mesh(),
        compiler_params=pltpu.CompilerParams(
            use_tc_tiling_on_sc=False, needs_layout_passes=False),
        scratch_types=[
            pltpu.VMEM((_SC_CH,), jnp.int32),
            pltpu.VMEM((_SC_CH + 16,), jnp.int32),
            pltpu.VMEM((_SC_CH + 16,), jnp.int32),
            pltpu.VMEM((2, _GB_S, H), jnp.float32),
            pltpu.VMEM((_R * H,), jnp.float32),
            pltpu.SemaphoreType.DMA,
        ],
    )
    def k(m_hbm, dst_hbm, out_hbm, dstbuf, idbuf, dstcbuf, rows_v, acc, sem):
        wid = lax.axis_index("s") * 2 + lax.axis_index("c")
        lo = wid * _R
        zf16 = jnp.zeros((16,), jnp.float32)
        zi16 = jnp.zeros((16,), jnp.int32)

        @pl.loop(0, _R * H // 16)
        def _(i):
            acc[pl.ds(i * 16, 16)] = zf16

        @pl.loop(0, (_SC_CH + 16) // 16)
        def _(i):
            idbuf[pl.ds(i * 16, 16)] = zi16

        iota16 = lax.broadcasted_iota(jnp.int32, (16,), 0)

        @pl.loop(0, _NSUP)
        def _(s):
            ebase = s * _SC_CH
            pltpu.sync_copy(dst_hbm.at[pl.ds(ebase, _SC_CH)], dstbuf)

            def scan_body(g, cnt):
                v = dstbuf[pl.ds(g * 16, 16)]
                mask = (v >= lo) & (v < lo + _R)
                ids = ebase + g * 16 + iota16
                plsc.store_compressed(idbuf.at[pl.ds(cnt, 16)], ids, mask=mask)
                plsc.store_compressed(dstcbuf.at[pl.ds(cnt, 16)], v, mask=mask)
                c = plsc.all_reduce_population_count(mask)
                return cnt + jnp.max(c)

            cnt = lax.fori_loop(0, _SC_CH // 16, scan_body, jnp.int32(0), unroll=4)

            nbat = (cnt + (_GB_S - 1)) // _GB_S

            @pl.when(nbat > 0)
            def _():
                pltpu.async_copy(
                    m_hbm.at[idbuf.at[pl.ds(0, _GB_S)]], rows_v.at[0], sem)

            def batch_body(gi, _):
                b = gi % 2
                goff = gi * _GB_S
                pltpu.make_async_copy(
                    m_hbm.at[idbuf.at[pl.ds(goff, _GB_S)]], rows_v.at[b], sem
                ).wait()

                @pl.when(gi + 1 < nbat)
                def _():
                    pltpu.async_copy(
                        m_hbm.at[idbuf.at[pl.ds(goff + _GB_S, _GB_S)]],
                        rows_v.at[1 - b], sem)

                nloc = jnp.minimum(_GB_S, cnt - goff)

                def acc_body(e, __):
                    d = dstcbuf[pl.ds(goff + e, 16)][0]
                    rel = (d - lo) * H
                    for c2 in range(H // 16):
                        off = rel + c2 * 16
                        a = acc[pl.ds(off, 16)]
                        r = rows_v[b, e, pl.ds(c2 * 16, 16)]
                        acc[pl.ds(off, 16)] = jnp.maximum(a, r)
                    return 0

                lax.fori_loop(0, nloc, acc_body, 0)
                return 0

            lax.fori_loop(0, nbat, batch_body, 0)

        pltpu.sync_copy(acc, out_hbm.at[pl.ds(wid * _R * H, _R * H)])

    return k(m, dst_s).reshape(NPAD, H)


_RT = 512
_CT = 1024
_KPAD = 32


def _knn(h_p, bc, br):
    nrt = NPAD // _RT

    def body(h_ref, bc_ref, br_ref, o_ref, buf_ref):
        INF = jnp.float32(jnp.inf)
        i = pl.program_id(0)
        hr = h_ref[pl.ds(i * _RT, _RT), :]
        sqr = jnp.sum(hr * hr, axis=1, keepdims=True)
        bcr = bc_ref[pl.ds(i * _RT, _RT), :]

        start_r = jnp.zeros((_RT, 1), jnp.int32)
        end_r = jnp.zeros((_RT, 1), jnp.int32)
        brow = br_ref[...]
        for b in range(NB):
            cb = jnp.sum(jnp.where(brow == b, 1, 0)).astype(jnp.int32)
            start_r = start_r + jnp.where(bcr > b, cb, 0)
            end_r = end_r + jnp.where(bcr >= b, cb, 0)
        tile_lo = jnp.min(start_r) // _CT
        tile_hi = (jnp.max(end_r) + _CT - 1) // _CT

        def dloop(t, _):
            hc = h_ref[pl.ds(t * _CT, _CT), :]
            sqc = jnp.sum(hc * hc, axis=1)[None, :]
            dot = lax.dot_general(
                hr, hc, (((1,), (1,)), ((), ())),
                preferred_element_type=jnp.float32,
            )
            d = (sqr + sqc) - 2.0 * dot
            mask = bcr == br_ref[pl.ds(0, 1), pl.ds(t * _CT, _CT)]
            buf_ref[:, pl.ds(t * _CT, _CT)] = jnp.where(mask, d, INF)
            return 0

        lax.fori_loop(tile_lo, tile_hi, dloop, 0)

        iota_c = lax.convert_element_type(
            lax.broadcasted_iota(jnp.int32, (_RT, _CT), 1), jnp.float32)
        BIGF = jnp.float32(2 * NPAD)
        prev = jnp.full((_RT, 1), BIGF)
        for kk in range(K):
            def eloop(t, carry):
                cm, ca = carry
                gidx = iota_c + lax.convert_element_type(t * _CT, jnp.float32)
                vals = buf_ref[:, pl.ds(t * _CT, _CT)]
                vals = jnp.where(gidx == prev, INF, vals)
                buf_ref[:, pl.ds(t * _CT, _CT)] = vals
                tmin = jnp.min(vals, axis=1, keepdims=True)
                targ = jnp.min(jnp.where(vals == tmin, gidx, BIGF),
                               axis=1, keepdims=True)
                upd = tmin < cm
                return (jnp.where(upd, tmin, cm), jnp.where(upd, targ, ca))

            cm, ca = lax.fori_loop(
                tile_lo, tile_hi, eloop,
                (jnp.full((_RT, 1), INF), jnp.full((_RT, 1), BIGF)),
            )
            o_ref[:, pl.ds(kk, 1)] = jnp.clip(
                lax.convert_element_type(ca, jnp.int32), 0, N - 1)
            prev = ca

    return pl.pallas_call(
        body,
        grid=(nrt,),
        in_specs=[
            pl.BlockSpec((NPAD, H), lambda i: (0, 0)),
            pl.BlockSpec((NPAD, 1), lambda i: (0, 0)),
            pl.BlockSpec((1, NPAD), lambda i: (0, 0)),
        ],
        out_specs=pl.BlockSpec((_RT, _KPAD), lambda i: (i, 0)),
        out_shape=jax.ShapeDtypeStruct((NPAD, _KPAD), jnp.int32),
        scratch_shapes=[pltpu.VMEM((_RT, NPAD), jnp.float32)],
    )(h_p, bc, br)


def _dyn_conv(g3, he, w1, w2, b1r, b2r, lwp, lbp, final):
    TB = 1024
    nt = NPAD // TB
    outw = 128 if final else H

    def body(g_ref, he_ref, w1_ref, w2_ref, b1_ref, b2_ref, lw_ref, lb_ref,
             o_ref, acc_ref):
        kk = pl.program_id(1)
        hi = he_ref[...]
        cat = jnp.concatenate([hi, g_ref[0] - hi], axis=1)
        h1 = jnp.dot(cat, w1_ref[...], preferred_element_type=jnp.float32) + b1_ref[...]
        h1 = jnp.maximum(h1, 0.0)
        h2 = jnp.dot(h1, w2_ref[...], preferred_element_type=jnp.float32) + b2_ref[...]
        h2 = jnp.maximum(h2, 0.0)

        @pl.when(kk == 0)
        def _():
            acc_ref[...] = h2

        @pl.when(kk > 0)
        def _():
            acc_ref[...] = jnp.maximum(acc_ref[...], h2)

        @pl.when(kk == K - 1)
        def _():
            if final:
                o_ref[...] = (
                    jnp.dot(acc_ref[...], lw_ref[...], preferred_element_type=jnp.float32)
                    + lb_ref[...]
                )
            else:
                o_ref[...] = acc_ref[...]

    return pl.pallas_call(
        body,
        grid=(nt, K),
        in_specs=[
            pl.BlockSpec((1, TB, H), lambda i, k: (k, i, 0)),
            pl.BlockSpec((TB, H), lambda i, k: (i, 0)),
            pl.BlockSpec((2 * H, H), lambda i, k: (0, 0)),
            pl.BlockSpec((H, H), lambda i, k: (0, 0)),
            pl.BlockSpec((1, H), lambda i, k: (0, 0)),
            pl.BlockSpec((1, H), lambda i, k: (0, 0)),
            pl.BlockSpec((H, 128), lambda i, k: (0, 0)),
            pl.BlockSpec((1, 128), lambda i, k: (0, 0)),
        ],
        out_specs=pl.BlockSpec((TB, outw), lambda i, k: (i, 0)),
        out_shape=jax.ShapeDtypeStruct((NPAD, outw), jnp.float32),
        scratch_shapes=[pltpu.VMEM((TB, H), jnp.float32)],
    )(g3, he, w1, w2, b1r, b2r, lwp, lbp)


def kernel(x, edge_index, batch, c1_W1, c1_b1, c1_W2, c1_b2,
           d1_W1, d1_b1, d1_W2, d1_b2, d2_W1, d2_b1, d2_W2, d2_b2,
           lin_W, lin_b):
    f32 = jnp.float32
    i32 = jnp.int32
    src = edge_index[0]
    dst = edge_index[1]
    e = src.shape[0]

    x_p = jnp.concatenate([x, jnp.zeros((NPAD - N, F), f32)], axis=0)
    batch_p = jnp.concatenate([batch.astype(i32), jnp.full((NPAD - N,), NB, i32)])
    bc = batch_p[:, None]
    br = batch_p[None, :]

    zpad = jnp.zeros((E_PAD - e,), i32)
    ids1 = jnp.concatenate([
        jnp.concatenate([dst, zpad]),
        jnp.concatenate([src, zpad]),
    ])
    gath1 = _sc_gather(x_p, ids1, 256)
    m1 = _mlp_edge(gath1, c1_W1, c1_W2, c1_b1.reshape(1, H), c1_b2.reshape(1, H))
    dst_s = jnp.concatenate([dst, jnp.full((E_PAD - e,), NPAD, i32)])
    h = _sc_scatter_max(m1, dst_s)

    lwp = jnp.concatenate([lin_W, jnp.zeros((H, 128 - C), f32)], axis=1)
    lbp = jnp.concatenate([lin_b, jnp.zeros((128 - C,), f32)]).reshape(1, 128)

    layers = ((d1_W1, d1_b1, d1_W2, d1_b2, False),
              (d2_W1, d2_b1, d2_W2, d2_b2, True))
    for (w1, b1, w2, b2, final) in layers:
        he = _elu(h)
        idxk = _knn(he, bc, br)
        ids2 = idxk[:, :K].T.reshape(K * NPAD)
        g = _sc_gather(he, ids2, 640)
        g3 = g.reshape(K, NPAD, H)
        h = _dyn_conv(g3, he, w1, w2, b1.reshape(1, H), b2.reshape(1, H),
                      lwp, lbp, final)

    return h[:N, :C]


# speedup vs baseline: 1.0436x; 1.0027x over previous
import functools

import jax
import jax.numpy as jnp
from jax import lax
from jax.experimental import pallas as pl
from jax.experimental.pallas import tpu as pltpu
from jax.experimental.pallas import tpu_sc as plsc

N = 10000
F = 128
H = 64
C = 40
K = 20
NB = 8

NPAD = 10240
E_PAD = 163840
NW = 32

def _sc_mesh():
    return plsc.VectorSubcoreMesh(core_axis_name="c", subcore_axis_name="s")


def _elu(h_p):
    def body(h_ref, o_ref):
        he = h_ref[...]
        o_ref[...] = jnp.where(he > 0, he, jnp.exp(he) - 1.0)

    return pl.pallas_call(
        body,
        out_shape=jax.ShapeDtypeStruct((NPAD, H), jnp.float32),
    )(h_p)


def _sc_gather(table, ids, gb):
    m = ids.shape[0]
    d = table.shape[1]
    per_w = m // NW
    nb = per_w // gb

    @functools.partial(
        pl.kernel,
        out_type=jax.ShapeDtypeStruct((m, d), jnp.float32),
        mesh=_sc_mesh(),
        compiler_params=pltpu.CompilerParams(use_tc_tiling_on_sc=False),
        scratch_types=[
            pltpu.VMEM((2, gb), jnp.int32),
            pltpu.VMEM((2, gb, d), jnp.float32),
            pltpu.SemaphoreType.DMA,
            pltpu.SemaphoreType.DMA,
            pltpu.SemaphoreType.DMA,
            pltpu.SemaphoreType.DMA,
        ],
    )
    def k(table_hbm, ids_hbm, out_hbm, idx_v, rows_v, gsa, gsb, wsa, wsb):
        wid = lax.axis_index("s") * 2 + lax.axis_index("c")
        base = wid * per_w

        def load_idx(i, buf):
            pltpu.sync_copy(ids_hbm.at[pl.ds(base + i * gb, gb)], idx_v.at[buf])

        load_idx(0, 0)
        pltpu.async_copy(table_hbm.at[idx_v.at[0]], rows_v.at[0], gsa)
        load_idx(1, 1)
        pltpu.async_copy(table_hbm.at[idx_v.at[1]], rows_v.at[1], gsb)

        @pl.loop(0, nb, step=2)
        def _(i):
            pltpu.make_async_copy(table_hbm.at[idx_v.at[0]], rows_v.at[0], gsa).wait()
            pltpu.async_copy(rows_v.at[0], out_hbm.at[pl.ds(base + i * gb, gb)], wsa)
            pltpu.make_async_copy(table_hbm.at[idx_v.at[1]], rows_v.at[1], gsb).wait()
            pltpu.async_copy(rows_v.at[1], out_hbm.at[pl.ds(base + (i + 1) * gb, gb)], wsb)

            @pl.when(i + 2 < nb)
            def _():
                load_idx(i + 2, 0)
                pltpu.make_async_copy(rows_v.at[0], out_hbm.at[pl.ds(0, gb)], wsa).wait()
                pltpu.async_copy(table_hbm.at[idx_v.at[0]], rows_v.at[0], gsa)

            @pl.when(i + 3 < nb)
            def _():
                load_idx(i + 3, 1)
                pltpu.make_async_copy(rows_v.at[1], out_hbm.at[pl.ds(0, gb)], wsb).wait()
                pltpu.async_copy(table_hbm.at[idx_v.at[1]], rows_v.at[1], gsb)

        pltpu.make_async_copy(rows_v.at[0], out_hbm.at[pl.ds(0, gb)], wsa).wait()
        pltpu.make_async_copy(rows_v.at[1], out_hbm.at[pl.ds(0, gb)], wsb).wait()

    return k(table, ids)


def _mlp_edge(gathered, w1, w2, b1r, b2r):
    TB = 2048
    nt = E_PAD // TB

    def body(xi_ref, xj_ref, w1_ref, w2_ref, b1_ref, b2_ref, o_ref):
        xi = xi_ref[...]
        cat = jnp.concatenate([xi, xj_ref[...] - xi], axis=1)
        h1 = jnp.dot(cat, w1_ref[...], preferred_element_type=jnp.float32) + b1_ref[...]
        h1 = jnp.maximum(h1, 0.0)
        h2 = jnp.dot(h1, w2_ref[...], preferred_element_type=jnp.float32) + b2_ref[...]
        o_ref[...] = jnp.maximum(h2, 0.0)

    return pl.pallas_call(
        body,
        grid=(nt,),
        in_specs=[
            pl.BlockSpec((TB, F), lambda i: (i, 0)),
            pl.BlockSpec((TB, F), lambda i: (i + nt, 0)),
            pl.BlockSpec((2 * F, H), lambda i: (0, 0)),
            pl.BlockSpec((H, H), lambda i: (0, 0)),
            pl.BlockSpec((1, H), lambda i: (0, 0)),
            pl.BlockSpec((1, H), lambda i: (0, 0)),
        ],
        out_specs=pl.BlockSpec((TB, H), lambda i: (i, 0)),
        out_shape=jax.ShapeDtypeStruct((E_PAD, H), jnp.float32),
    )(gathered, gathered, w1, w2, b1r, b2r)


_R = NPAD // NW
_SC_CH = 16384
_GB_S = 256
_NSUP = E_PAD // _SC_CH


def _sc_scatter_max(m, dst_s):
    @functools.partial(
        pl.kernel,
        out_type=jax.ShapeDtypeStruct((NPAD * H,), jnp.float32),
        mesh=_sc_mesh(),
        compiler_params=pltpu.CompilerParams(
            use_tc_tiling_on_sc=False, needs_layout_passes=False),
        scratch_types=[
            pltpu.VMEM((_SC_CH,), jnp.int32),
            pltpu.VMEM((_SC_CH + 16,), jnp.int32),
            pltpu.VMEM((_SC_CH + 16,), jnp.int32),
            pltpu.VMEM((2, _GB_S, H), jnp.float32),
            pltpu.VMEM((_R * H,), jnp.float32),
            pltpu.SemaphoreType.DMA,
        ],
    )
    def k(m_hbm, dst_hbm, out_hbm, dstbuf, idbuf, dstcbuf, rows_v, acc, sem):
        wid = lax.axis_index("s") * 2 + lax.axis_index("c")
        lo = wid * _R
        zf16 = jnp.zeros((16,), jnp.float32)
        zi16 = jnp.zeros((16,), jnp.int32)

        @pl.loop(0, _R * H // 16)
        def _(i):
            acc[pl.ds(i * 16, 16)] = zf16

        @pl.loop(0, (_SC_CH + 16) // 16)
        def _(i):
            idbuf[pl.ds(i * 16, 16)] = zi16

        iota16 = lax.broadcasted_iota(jnp.int32, (16,), 0)

        @pl.loop(0, _NSUP)
        def _(s):
            ebase = s * _SC_CH
            pltpu.sync_copy(dst_hbm.at[pl.ds(ebase, _SC_CH)], dstbuf)

            def scan_body(g, cnt):
                v = dstbuf[pl.ds(g * 16, 16)]
                mask = (v >= lo) & (v < lo + _R)
                ids = ebase + g * 16 + iota16
                plsc.store_compressed(idbuf.at[pl.ds(cnt, 16)], ids, mask=mask)
                plsc.store_compressed(dstcbuf.at[pl.ds(cnt, 16)], v, mask=mask)
                c = plsc.all_reduce_population_count(mask)
                return cnt + c[0]

            cnt = lax.fori_loop(0, _SC_CH // 16, scan_body, jnp.int32(0), unroll=4)

            nbat = (cnt + (_GB_S - 1)) // _GB_S

            @pl.when(nbat > 0)
            def _():
                pltpu.async_copy(
                    m_hbm.at[idbuf.at[pl.ds(0, _GB_S)]], rows_v.at[0], sem)

            def batch_body(gi, _):
                b = gi % 2
                goff = gi * _GB_S
                pltpu.make_async_copy(
                    m_hbm.at[idbuf.at[pl.ds(goff, _GB_S)]], rows_v.at[b], sem
                ).wait()

                @pl.when(gi + 1 < nbat)
                def _():
                    pltpu.async_copy(
                        m_hbm.at[idbuf.at[pl.ds(goff + _GB_S, _GB_S)]],
                        rows_v.at[1 - b], sem)

                nloc = jnp.minimum(_GB_S, cnt - goff)

                def acc_body(e, __):
                    d = dstcbuf[pl.ds(goff + e, 16)][0]
                    rel = (d - lo) * H
                    for c2 in range(H // 16):
                        off = rel + c2 * 16
                        a = acc[pl.ds(off, 16)]
                        r = rows_v[b, e, pl.ds(c2 * 16, 16)]
                        acc[pl.ds(off, 16)] = jnp.maximum(a, r)
                    return 0

                lax.fori_loop(0, nloc, acc_body, 0)
                return 0

            lax.fori_loop(0, nbat, batch_body, 0)

        pltpu.sync_copy(acc, out_hbm.at[pl.ds(wid * _R * H, _R * H)])

    return k(m, dst_s).reshape(NPAD, H)


_RT = 512
_CT = 1024
_KPAD = 32


def _knn(h_p, bc, br):
    nrt = NPAD // _RT

    def body(h_ref, bc_ref, br_ref, o_ref, buf_ref):
        INF = jnp.float32(jnp.inf)
        i = pl.program_id(0)
        hr = h_ref[pl.ds(i * _RT, _RT), :]
        sqr = jnp.sum(hr * hr, axis=1, keepdims=True)
        bcr = bc_ref[pl.ds(i * _RT, _RT), :]

        start_r = jnp.zeros((_RT, 1), jnp.int32)
        end_r = jnp.zeros((_RT, 1), jnp.int32)
        brow = br_ref[...]
        for b in range(NB):
            cb = jnp.sum(jnp.where(brow == b, 1, 0)).astype(jnp.int32)
            start_r = start_r + jnp.where(bcr > b, cb, 0)
            end_r = end_r + jnp.where(bcr >= b, cb, 0)
        tile_lo = jnp.min(start_r) // _CT
        tile_hi = (jnp.max(end_r) + _CT - 1) // _CT

        def dloop(t, _):
            hc = h_ref[pl.ds(t * _CT, _CT), :]
            sqc = jnp.sum(hc * hc, axis=1)[None, :]
            dot = lax.dot_general(
                hr, hc, (((1,), (1,)), ((), ())),
                preferred_element_type=jnp.float32,
            )
            d = (sqr + sqc) - 2.0 * dot
            mask = bcr == br_ref[pl.ds(0, 1), pl.ds(t * _CT, _CT)]
            buf_ref[:, pl.ds(t * _CT, _CT)] = jnp.where(mask, d, INF)
            return 0

        lax.fori_loop(tile_lo, tile_hi, dloop, 0)

        iota_c = lax.convert_element_type(
            lax.broadcasted_iota(jnp.int32, (_RT, _CT), 1), jnp.float32)
        BIGF = jnp.float32(2 * NPAD)
        prev = jnp.full((_RT, 1), BIGF)
        for kk in range(K):
            def eloop(t, carry):
                cm, ca = carry
                gidx = iota_c + lax.convert_element_type(t * _CT, jnp.float32)
                vals = buf_ref[:, pl.ds(t * _CT, _CT)]
                vals = jnp.where(gidx == prev, INF, vals)
                buf_ref[:, pl.ds(t * _CT, _CT)] = vals
                tmin = jnp.min(vals, axis=1, keepdims=True)
                targ = jnp.min(jnp.where(vals == tmin, gidx, BIGF),
                               axis=1, keepdims=True)
                upd = tmin < cm
                return (jnp.where(upd, tmin, cm), jnp.where(upd, targ, ca))

            cm, ca = lax.fori_loop(
                tile_lo, tile_hi, eloop,
                (jnp.full((_RT, 1), INF), jnp.full((_RT, 1), BIGF)),
            )
            o_ref[:, pl.ds(kk, 1)] = jnp.clip(
                lax.convert_element_type(ca, jnp.int32), 0, N - 1)
            prev = ca

    return pl.pallas_call(
        body,
        grid=(nrt,),
        in_specs=[
            pl.BlockSpec((NPAD, H), lambda i: (0, 0)),
            pl.BlockSpec((NPAD, 1), lambda i: (0, 0)),
            pl.BlockSpec((1, NPAD), lambda i: (0, 0)),
        ],
        out_specs=pl.BlockSpec((_RT, _KPAD), lambda i: (i, 0)),
        out_shape=jax.ShapeDtypeStruct((NPAD, _KPAD), jnp.int32),
        scratch_shapes=[pltpu.VMEM((_RT, NPAD), jnp.float32)],
    )(h_p, bc, br)


def _dyn_conv(g3, he, w1, w2, b1r, b2r, lwp, lbp, final):
    TB = 1024
    nt = NPAD // TB
    outw = 128 if final else H

    def body(g_ref, he_ref, w1_ref, w2_ref, b1_ref, b2_ref, lw_ref, lb_ref,
             o_ref, acc_ref):
        kk = pl.program_id(1)
        hi = he_ref[...]
        cat = jnp.concatenate([hi, g_ref[0] - hi], axis=1)
        h1 = jnp.dot(cat, w1_ref[...], preferred_element_type=jnp.float32) + b1_ref[...]
        h1 = jnp.maximum(h1, 0.0)
        h2 = jnp.dot(h1, w2_ref[...], preferred_element_type=jnp.float32) + b2_ref[...]
        h2 = jnp.maximum(h2, 0.0)

        @pl.when(kk == 0)
        def _():
            acc_ref[...] = h2

        @pl.when(kk > 0)
        def _():
            acc_ref[...] = jnp.maximum(acc_ref[...], h2)

        @pl.when(kk == K - 1)
        def _():
            if final:
                o_ref[...] = (
                    jnp.dot(acc_ref[...], lw_ref[...], preferred_element_type=jnp.float32)
                    + lb_ref[...]
                )
            else:
                o_ref[...] = acc_ref[...]

    return pl.pallas_call(
        body,
        grid=(nt, K),
        in_specs=[
            pl.BlockSpec((1, TB, H), lambda i, k: (k, i, 0)),
            pl.BlockSpec((TB, H), lambda i, k: (i, 0)),
            pl.BlockSpec((2 * H, H), lambda i, k: (0, 0)),
            pl.BlockSpec((H, H), lambda i, k: (0, 0)),
            pl.BlockSpec((1, H), lambda i, k: (0, 0)),
            pl.BlockSpec((1, H), lambda i, k: (0, 0)),
            pl.BlockSpec((H, 128), lambda i, k: (0, 0)),
            pl.BlockSpec((1, 128), lambda i, k: (0, 0)),
        ],
        out_specs=pl.BlockSpec((TB, outw), lambda i, k: (i, 0)),
        out_shape=jax.ShapeDtypeStruct((NPAD, outw), jnp.float32),
        scratch_shapes=[pltpu.VMEM((TB, H), jnp.float32)],
    )(g3, he, w1, w2, b1r, b2r, lwp, lbp)


def kernel(x, edge_index, batch, c1_W1, c1_b1, c1_W2, c1_b2,
           d1_W1, d1_b1, d1_W2, d1_b2, d2_W1, d2_b1, d2_W2, d2_b2,
           lin_W, lin_b):
    f32 = jnp.float32
    i32 = jnp.int32
    src = edge_index[0]
    dst = edge_index[1]
    e = src.shape[0]

    x_p = jnp.concatenate([x, jnp.zeros((NPAD - N, F), f32)], axis=0)
    batch_p = jnp.concatenate([batch.astype(i32), jnp.full((NPAD - N,), NB, i32)])
    bc = batch_p[:, None]
    br = batch_p[None, :]

    zpad = jnp.zeros((E_PAD - e,), i32)
    ids1 = jnp.concatenate([
        jnp.concatenate([dst, zpad]),
        jnp.concatenate([src, zpad]),
    ])
    gath1 = _sc_gather(x_p, ids1, 256)
    m1 = _mlp_edge(gath1, c1_W1, c1_W2, c1_b1.reshape(1, H), c1_b2.reshape(1, H))
    dst_s = jnp.concatenate([dst, jnp.full((E_PAD - e,), NPAD, i32)])
    h = _sc_scatter_max(m1, dst_s)

    lwp = jnp.concatenate([lin_W, jnp.zeros((H, 128 - C), f32)], axis=1)
    lbp = jnp.concatenate([lin_b, jnp.zeros((128 - C,), f32)]).reshape(1, 128)

    layers = ((d1_W1, d1_b1, d1_W2, d1_b2, False),
              (d2_W1, d2_b1, d2_W2, d2_b2, True))
    for (w1, b1, w2, b2, final) in layers:
        he = _elu(h)
        idxk = _knn(he, bc, br)
        ids2 = idxk[:, :K].T.reshape(K * NPAD)
        g = _sc_gather(he, ids2, 640)
        g3 = g.reshape(K, NPAD, H)
        h = _dyn_conv(g3, he, w1, w2, b1.reshape(1, H), b2.reshape(1, H),
                      lwp, lbp, final)

    return h[:N, :C]
